# Initial kernel scaffold; baseline (speedup 1.0000x reference)
import jax
import jax.numpy as jnp
from jax.experimental import pallas as pl


def kernel(time, emb_weight):
    raise NotImplementedError("write your pallas kernel here")


# speedup vs baseline: 6.4845x; 6.4845x over previous
import functools

import jax
import jax.numpy as jnp
from jax import lax
from jax.experimental import pallas as pl
from jax.experimental.pallas import tpu as pltpu
from jax.experimental.pallas import tpu_sc as plsc

NC = 2
NS = 16
NW = NC * NS
CHUNK = 128
NBUF = 4


def _emb_lookup(idx, table, total, D):
    nchunks_w = (total // NW) // CHUNK
    per_w = total // NW
    mesh = plsc.VectorSubcoreMesh(core_axis_name="c", subcore_axis_name="s")

    @functools.partial(
        pl.kernel,
        out_type=jax.ShapeDtypeStruct((total, D), jnp.float32),
        mesh=mesh,
        scratch_types=[
            pltpu.VMEM((nchunks_w, CHUNK), jnp.int32),
            pltpu.VMEM((NBUF, CHUNK, D), jnp.float32),
        ]
        + [pltpu.SemaphoreType.DMA] * (2 * NBUF),
    )
    def body(idx_hbm, table_hbm, out_hbm, idx_v, bufs, *sems):
        gsem = sems[:NBUF]
        wsem = sems[NBUF:]
        wid = lax.axis_index("c") * NS + lax.axis_index("s")
        base = wid * per_w
        pltpu.sync_copy(idx_hbm.at[wid], idx_v)

        def start_gather(j, b):
            pltpu.async_copy(table_hbm.at[idx_v.at[j]], bufs.at[b], gsem[b])

        def wait_gather(b):
            pltpu.make_async_copy(
                table_hbm.at[pl.ds(0, CHUNK)], bufs.at[b], gsem[b]
            ).wait()

        def start_write(j, b):
            pltpu.async_copy(
                bufs.at[b], out_hbm.at[pl.ds(base + j * CHUNK, CHUNK)], wsem[b]
            )

        def wait_write(b):
            pltpu.make_async_copy(
                bufs.at[b], out_hbm.at[pl.ds(base, CHUNK)], wsem[b]
            ).wait()

        for b in range(NBUF):
            start_gather(b, b)

        ngroups = nchunks_w // NBUF

        @pl.loop(0, ngroups - 1)
        def _(g):
            j0 = g * NBUF
            for b in range(NBUF):
                wait_gather(b)
                start_write(j0 + b, b)
            for b in range(NBUF):
                wait_write(b)
                start_gather(j0 + NBUF + b, b)

        j0 = (ngroups - 1) * NBUF
        for b in range(NBUF):
            wait_gather(b)
            start_write(j0 + b, b)
        for b in range(NBUF):
            wait_write(b)

    return body(idx, table)


def kernel(time, emb_weight):
    B, H = time.shape
    V, D = emb_weight.shape
    total = B * H
    idx = time.reshape(NW, (total // NW) // CHUNK, CHUNK).astype(jnp.int32)
    out = _emb_lookup(idx, emb_weight, total, D)
    return out.reshape(B, H, D)


# speedup vs baseline: 15.6770x; 2.4176x over previous
import functools

import jax
import jax.numpy as jnp
from jax import lax
from jax.experimental import pallas as pl
from jax.experimental.pallas import tpu as pltpu
from jax.experimental.pallas import tpu_sc as plsc

NC = 2
NS = 16
NW = NC * NS
CHUNK = 128
NBUF = 4


def _emb_lookup(idx, table, total, D, V):
    per_w = total // NW
    nchunks_w = per_w // CHUNK
    mesh = plsc.VectorSubcoreMesh(core_axis_name="c", subcore_axis_name="s")

    @functools.partial(
        pl.kernel,
        out_type=jax.ShapeDtypeStruct((total, D), jnp.float32),
        mesh=mesh,
        scratch_types=[
            pltpu.VMEM((nchunks_w, CHUNK), jnp.int32),
            pltpu.VMEM((NBUF, CHUNK, D), jnp.float32),
            pltpu.VMEM_SHARED((V, D), jnp.float32),
        ]
        + [pltpu.SemaphoreType.DMA] * (2 * NBUF),
    )
    def body(idx_hbm, table_hbm, out_hbm, idx_v, bufs, table_sp, *sems):
        gsem = sems[:NBUF]
        wsem = sems[NBUF:]
        sid = lax.axis_index("s")
        wid = lax.axis_index("c") * NS + sid
        base = wid * per_w

        @pl.when(sid == 0)
        def _():
            pltpu.sync_copy(table_hbm, table_sp)

        pltpu.sync_copy(idx_hbm.at[wid], idx_v)
        plsc.subcore_barrier()

        def start_gather(j, b):
            pltpu.async_copy(table_sp.at[idx_v.at[j]], bufs.at[b], gsem[b])

        def wait_gather(b):
            pltpu.make_async_copy(
                table_sp.at[pl.ds(0, CHUNK)], bufs.at[b], gsem[b]
            ).wait()

        def start_write(j, b):
            pltpu.async_copy(
                bufs.at[b], out_hbm.at[pl.ds(base + j * CHUNK, CHUNK)], wsem[b]
            )

        def wait_write(b):
            pltpu.make_async_copy(
                bufs.at[b], out_hbm.at[pl.ds(base, CHUNK)], wsem[b]
            ).wait()

        for b in range(NBUF):
            start_gather(b, b)

        ngroups = nchunks_w // NBUF

        @pl.loop(0, ngroups - 1)
        def _(g):
            j0 = g * NBUF
            for b in range(NBUF):
                wait_gather(b)
                start_write(j0 + b, b)
            for b in range(NBUF):
                wait_write(b)
                start_gather(j0 + NBUF + b, b)

        j0 = (ngroups - 1) * NBUF
        for b in range(NBUF):
            wait_gather(b)
            start_write(j0 + b, b)
        for b in range(NBUF):
            wait_write(b)

    return body(idx, table)


def kernel(time, emb_weight):
    B, H = time.shape
    V, D = emb_weight.shape
    total = B * H
    idx = time.reshape(NW, (total // NW) // CHUNK, CHUNK).astype(jnp.int32)
    out = _emb_lookup(idx, emb_weight, total, D, V)
    return out.reshape(B, H, D)
